# Initial kernel scaffold; baseline (speedup 1.0000x reference)
import jax
import jax.numpy as jnp
from jax.experimental import pallas as pl


def kernel(x, edge_index, node_mask_logit):
    raise NotImplementedError("write your pallas kernel here")


# speedup vs baseline: 18.0799x; 18.0799x over previous
import functools

import jax
import jax.numpy as jnp
from jax import lax
from jax.experimental import pallas as pl
from jax.experimental.pallas import tpu as pltpu
from jax.experimental.pallas import tpu_sc as plsc

NC = 2
NS = 16
NW = NC * NS
BLK = 128
PACK_SHIFT = 14


def _hard_mask(node_mask_logit, n, dtype):
    node_mask_p = jax.nn.sigmoid(node_mask_logit)
    logits = jnp.log(jnp.concatenate([node_mask_p, 1.0 - node_mask_p], axis=0))
    u = jax.random.uniform(jax.random.key(42), logits.shape,
                           minval=1e-10, maxval=1.0)
    g = -jnp.log(-jnp.log(u))
    y_soft = jax.nn.softmax((logits + g) / 1.0, axis=0)
    idx = jnp.argmax(y_soft, axis=0)
    y_hard = jax.nn.one_hot(idx, logits.shape[0], axis=0, dtype=y_soft.dtype)
    y = y_hard + y_soft - jax.lax.stop_gradient(y_soft)
    return jnp.concatenate([y[0], jnp.ones((1,), dtype=dtype)])


def _make_sc_kernel(n, d, bpt, n_words, acc_rows, rows_per_tile):
    mesh = plsc.VectorSubcoreMesh(core_axis_name="c", subcore_axis_name="s")

    @functools.partial(
        pl.kernel,
        mesh=mesh,
        compiler_params=pltpu.CompilerParams(needs_layout_passes=False),
        out_type=jax.ShapeDtypeStruct((NC, acc_rows, d), jnp.float32),
        scratch_types=[
            pltpu.VMEM((n_words,), jnp.int32),
            pltpu.VMEM((bpt, BLK), jnp.int32),
            pltpu.VMEM((2, BLK), jnp.int32),
            pltpu.VMEM((2, BLK), jnp.int32),
            pltpu.VMEM((2, BLK, d), jnp.float32),
            pltpu.VMEM_SHARED((acc_rows, d), jnp.float32),
            pltpu.SemaphoreType.DMA((2,)),
            pltpu.SemaphoreType.DMA((2,)),
        ],
    )
    def sc_kernel(mask_hbm, pck_hbm, x_hbm, outp_hbm,
                  mask_v, pck_v, sstg_v, dstg_v, rows_v, acc_sh,
                  sem_g, sem_s):
        cid = lax.axis_index("c")
        sid = lax.axis_index("s")
        wid = sid * NC + cid
        trash = n + sid

        pltpu.sync_copy(mask_hbm, mask_v)
        pltpu.sync_copy(pck_hbm.at[wid], pck_v)

        zero16 = jnp.zeros((16,), jnp.float32)
        for i in range(8):
            for j in range(d // 16):
                rows_v[0, i, pl.ds(j * 16, 16)] = zero16
        base = sid * rows_per_tile

        def zero_body(k, _):
            pltpu.sync_copy(rows_v.at[0, pl.ds(0, 8)],
                            acc_sh.at[pl.ds(base + k * 8, 8)])
            return 0
        lax.fori_loop(0, rows_per_tile // 8, zero_body, 0)
        plsc.subcore_barrier()

        def unpack_block(b, slot):
            for g in range(BLK // 16):
                p16 = pck_v[b, pl.ds(g * 16, 16)]
                s16 = p16 & ((1 << PACK_SHIFT) - 1)
                d16 = lax.shift_right_logical(p16, PACK_SHIFT)
                ws = plsc.load_gather(mask_v, [lax.shift_right_logical(s16, 5)])
                wd = plsc.load_gather(mask_v, [lax.shift_right_logical(d16, 5)])
                bs = lax.shift_right_logical(ws, s16 & 31)
                bd = lax.shift_right_logical(wd, d16 & 31)
                keep = (bs & bd & 1) == 1
                sstg_v[slot, pl.ds(g * 16, 16)] = s16
                dstg_v[slot, pl.ds(g * 16, 16)] = jnp.where(keep, d16, trash)

        unpack_block(0, 0)
        pltpu.make_async_copy(
            x_hbm.at[sstg_v.at[0]], rows_v.at[0], sem_g.at[0]).start()

        def block_body(b, _):
            slot = lax.rem(b, 2)
            other = 1 - slot
            pltpu.make_async_copy(
                x_hbm.at[sstg_v.at[slot]], rows_v.at[slot],
                sem_g.at[slot]).wait()

            @pl.when(b >= 1)
            def _():
                pltpu.make_async_copy(
                    rows_v.at[other], acc_sh.at[dstg_v.at[other]],
                    sem_s.at[other]).wait()

            @pl.when(b + 1 < bpt)
            def _():
                unpack_block(b + 1, other)
                pltpu.make_async_copy(
                    x_hbm.at[sstg_v.at[other]], rows_v.at[other],
                    sem_g.at[other]).start()

            pltpu.async_copy(rows_v.at[slot], acc_sh.at[dstg_v.at[slot]],
                             sem_s.at[slot], add=True)
            return 0

        lax.fori_loop(0, bpt, block_body, 0)
        last = (bpt - 1) % 2
        pltpu.make_async_copy(
            rows_v.at[last], acc_sh.at[dstg_v.at[last]], sem_s.at[last]).wait()
        plsc.subcore_barrier()

        pltpu.sync_copy(acc_sh.at[pl.ds(base, rows_per_tile)],
                        outp_hbm.at[cid, pl.ds(base, rows_per_tile)])

    return sc_kernel


def _tc_combine(partials, n, d, blk_rows):
    def body(p_ref, o_ref):
        o_ref[...] = p_ref[0] + p_ref[1]

    return pl.pallas_call(
        body,
        grid=(n // blk_rows,),
        in_specs=[pl.BlockSpec((NC, blk_rows, d), lambda i: (0, i, 0))],
        out_specs=pl.BlockSpec((blk_rows, d), lambda i: (i, 0)),
        out_shape=jax.ShapeDtypeStruct((n, d), jnp.float32),
    )(partials)


def kernel(x, edge_index, node_mask_logit):
    n, d = x.shape
    e = edge_index.shape[1]

    hard_mask = _hard_mask(node_mask_logit, n, x.dtype)

    bpt = -(-e // (NW * BLK))
    e_pad = NW * bpt * BLK
    n_bits = -(-(n + 1) // 32) * 32
    n_words = max(n_bits // 32, 8)
    rows_per_tile = -(-(n + NS) // NS // 8) * 8
    acc_rows = rows_per_tile * NS

    src = edge_index[0]
    dst = edge_index[1]
    pad = e_pad - e
    src_p = jnp.concatenate([src, jnp.zeros((pad,), jnp.int32)])
    dst_p = jnp.concatenate([dst, jnp.full((pad,), n, jnp.int32)])
    packed = (src_p | (dst_p << PACK_SHIFT)).reshape(NW, bpt, BLK)

    bits = jnp.concatenate(
        [(hard_mask > 0.5).astype(jnp.uint32),
         jnp.zeros((n_words * 32 - n,), jnp.uint32)])
    words = (bits.reshape(n_words, 32) << jnp.arange(32, dtype=jnp.uint32)
             ).sum(axis=1, dtype=jnp.uint32).view(jnp.int32)

    sc = _make_sc_kernel(n, d, bpt, n_words, acc_rows, rows_per_tile)
    partials = sc(words, packed, x)

    blk_rows = 2000 if n % 2000 == 0 else 8
    return _tc_combine(partials, n, d, blk_rows)


# speedup vs baseline: 35.6454x; 1.9715x over previous
import functools

import jax
import jax.numpy as jnp
from jax import lax
from jax.experimental import pallas as pl
from jax.experimental.pallas import tpu as pltpu
from jax.experimental.pallas import tpu_sc as plsc

NC = 2
NS = 16
NW = NC * NS
BLK = 128
PACK_SHIFT = 14


def _hard_mask(node_mask_logit, n, dtype):
    node_mask_p = jax.nn.sigmoid(node_mask_logit)
    logits = jnp.log(jnp.concatenate([node_mask_p, 1.0 - node_mask_p], axis=0))
    u = jax.random.uniform(jax.random.key(42), logits.shape,
                           minval=1e-10, maxval=1.0)
    g = -jnp.log(-jnp.log(u))
    y_soft = jax.nn.softmax((logits + g) / 1.0, axis=0)
    idx = jnp.argmax(y_soft, axis=0)
    y_hard = jax.nn.one_hot(idx, logits.shape[0], axis=0, dtype=y_soft.dtype)
    y = y_hard + y_soft - jax.lax.stop_gradient(y_soft)
    return jnp.concatenate([y[0], jnp.ones((1,), dtype=dtype)])


def _make_sc_kernel(n, d, bpt, n_words, acc_rows, rows_per_tile):
    mesh = plsc.VectorSubcoreMesh(core_axis_name="c", subcore_axis_name="s")
    epw = bpt * BLK

    @functools.partial(
        pl.kernel,
        mesh=mesh,
        compiler_params=pltpu.CompilerParams(needs_layout_passes=False),
        out_type=jax.ShapeDtypeStruct((NC, acc_rows, d), jnp.float32),
        scratch_types=[
            pltpu.VMEM((n_words,), jnp.int32),
            pltpu.VMEM((epw,), jnp.int32),
            pltpu.VMEM((2, BLK), jnp.int32),
            pltpu.VMEM((2, BLK), jnp.int32),
            pltpu.VMEM((2, BLK, d), jnp.float32),
            pltpu.VMEM_SHARED((acc_rows, d), jnp.float32),
            pltpu.SemaphoreType.DMA((2,)),
            pltpu.SemaphoreType.DMA((2,)),
        ],
    )
    def sc_kernel(mask_hbm, pck_hbm, x_hbm, outp_hbm,
                  mask_v, pck_v, sstg_v, dstg_v, rows_v, acc_sh,
                  sem_g, sem_s):
        cid = lax.axis_index("c")
        sid = lax.axis_index("s")
        wid = sid * NC + cid
        trash = n + sid

        pltpu.sync_copy(mask_hbm, mask_v)
        pltpu.sync_copy(pck_hbm.at[wid], pck_v)

        zero16 = jnp.zeros((16,), jnp.float32)
        for i in range(8):
            for j in range(d // 16):
                rows_v[0, i, pl.ds(j * 16, 16)] = zero16
        base = sid * rows_per_tile

        def zero_body(k, _):
            pltpu.async_copy(rows_v.at[0, pl.ds(0, 8)],
                             acc_sh.at[pl.ds(base + k * 8, 8)], sem_g.at[0])
            return 0
        lax.fori_loop(0, rows_per_tile // 8, zero_body, 0)

        def zero_drain(k, _):
            pltpu.make_async_copy(rows_v.at[0, pl.ds(0, 8)],
                                  acc_sh.at[pl.ds(base + k * 8, 8)],
                                  sem_g.at[0]).wait()
            return 0
        lax.fori_loop(0, rows_per_tile // 8, zero_drain, 0)

        def compact_group(i, cnt):
            p16 = pck_v[pl.ds(i * 16, 16)]
            s16 = p16 & ((1 << PACK_SHIFT) - 1)
            d16 = lax.shift_right_logical(p16, PACK_SHIFT)
            ws = plsc.load_gather(mask_v, [lax.shift_right_logical(s16, 5)])
            wd = plsc.load_gather(mask_v, [lax.shift_right_logical(d16, 5)])
            bs = lax.shift_right_logical(ws, s16 & 31)
            bd = lax.shift_right_logical(wd, d16 & 31)
            keep = (bs & bd & 1) == 1
            plsc.store_compressed(pck_v.at[pl.ds(cnt, 16)], p16, mask=keep)
            return cnt + jnp.max(plsc.all_reduce_population_count(keep))

        cnt = lax.fori_loop(0, epw // 16, compact_group, jnp.int32(0))

        padval = jnp.full((16,), trash << PACK_SHIFT, jnp.int32)
        for g in range(BLK // 16):
            pck_v[pl.ds(cnt + g * 16, 16)] = padval
        nblk = jnp.maximum((cnt + BLK - 1) // BLK, 1)

        plsc.subcore_barrier()

        def stage_block(b, slot):
            for g in range(BLK // 16):
                p16 = pck_v[pl.ds(b * BLK + g * 16, 16)]
                sstg_v[slot, pl.ds(g * 16, 16)] = p16 & ((1 << PACK_SHIFT) - 1)
                dstg_v[slot, pl.ds(g * 16, 16)] = lax.shift_right_logical(
                    p16, PACK_SHIFT)

        stage_block(0, 0)
        pltpu.make_async_copy(
            x_hbm.at[sstg_v.at[0]], rows_v.at[0], sem_g.at[0]).start()

        def block_body(b, _):
            slot = lax.rem(b, 2)
            other = 1 - slot
            pltpu.make_async_copy(
                x_hbm.at[sstg_v.at[slot]], rows_v.at[slot],
                sem_g.at[slot]).wait()

            @pl.when(b >= 1)
            def _():
                pltpu.make_async_copy(
                    rows_v.at[other], acc_sh.at[dstg_v.at[other]],
                    sem_s.at[other]).wait()

            pltpu.async_copy(rows_v.at[slot], acc_sh.at[dstg_v.at[slot]],
                             sem_s.at[slot], add=True)

            @pl.when(b + 1 < nblk)
            def _():
                stage_block(b + 1, other)
                pltpu.make_async_copy(
                    x_hbm.at[sstg_v.at[other]], rows_v.at[other],
                    sem_g.at[other]).start()
            return 0

        lax.fori_loop(0, nblk, block_body, 0)
        last = lax.rem(nblk - 1, 2)

        def drain_last(slot):
            pltpu.make_async_copy(
                rows_v.at[slot], acc_sh.at[dstg_v.at[slot]],
                sem_s.at[slot]).wait()

        @pl.when(last == 0)
        def _():
            drain_last(0)

        @pl.when(last == 1)
        def _():
            drain_last(1)

        plsc.subcore_barrier()

        pltpu.sync_copy(acc_sh.at[pl.ds(base, rows_per_tile)],
                        outp_hbm.at[cid, pl.ds(base, rows_per_tile)])

    return sc_kernel


def _tc_combine(partials, n, d, blk_rows):
    def body(p_ref, o_ref):
        o_ref[...] = p_ref[0] + p_ref[1]

    return pl.pallas_call(
        body,
        grid=(n // blk_rows,),
        in_specs=[pl.BlockSpec((NC, blk_rows, d), lambda i: (0, i, 0))],
        out_specs=pl.BlockSpec((blk_rows, d), lambda i: (i, 0)),
        out_shape=jax.ShapeDtypeStruct((n, d), jnp.float32),
    )(partials)


def kernel(x, edge_index, node_mask_logit):
    n, d = x.shape
    e = edge_index.shape[1]

    hard_mask = _hard_mask(node_mask_logit, n, x.dtype)

    bpt = -(-e // (NW * BLK)) + 1
    e_pad = NW * bpt * BLK
    n_bits = -(-(n + 1) // 32) * 32
    n_words = -(-(n_bits // 32) // 16) * 16
    rows_per_tile = -(-(n + NS) // NS // 8) * 8
    acc_rows = rows_per_tile * NS

    src = edge_index[0]
    dst = edge_index[1]
    pad = e_pad - e
    src_p = jnp.concatenate([src, jnp.zeros((pad,), jnp.int32)])
    dst_p = jnp.concatenate([dst, jnp.full((pad,), n, jnp.int32)])
    packed = (src_p | (dst_p << PACK_SHIFT)).reshape(NW, bpt * BLK)

    bits = jnp.concatenate(
        [(hard_mask > 0.5).astype(jnp.uint32),
         jnp.zeros((n_words * 32 - n,), jnp.uint32)])
    words = (bits.reshape(n_words, 32) << jnp.arange(32, dtype=jnp.uint32)
             ).sum(axis=1, dtype=jnp.uint32).view(jnp.int32)

    sc = _make_sc_kernel(n, d, bpt, n_words, acc_rows, rows_per_tile)
    partials = sc(words, packed, x)

    blk_rows = 2000 if n % 2000 == 0 else 8
    return _tc_combine(partials, n, d, blk_rows)


# speedup vs baseline: 49.9726x; 1.4019x over previous
import functools

import jax
import jax.numpy as jnp
from jax import lax
from jax.experimental import pallas as pl
from jax.experimental.pallas import tpu as pltpu
from jax.experimental.pallas import tpu_sc as plsc

NC = 2
NS = 16
NW = NC * NS
BLK = 128
PACK_SHIFT = 14
GBLK = 64


def _hard_mask(node_mask_logit, n, dtype):
    node_mask_p = jax.nn.sigmoid(node_mask_logit)
    logits = jnp.log(jnp.concatenate([node_mask_p, 1.0 - node_mask_p], axis=0))
    u = jax.random.uniform(jax.random.key(42), logits.shape,
                           minval=1e-10, maxval=1.0)
    g = -jnp.log(-jnp.log(u))
    y_soft = jax.nn.softmax((logits + g) / 1.0, axis=0)
    idx = jnp.argmax(y_soft, axis=0)
    y_hard = jax.nn.one_hot(idx, logits.shape[0], axis=0, dtype=y_soft.dtype)
    y = y_hard + y_soft - jax.lax.stop_gradient(y_soft)
    return jnp.concatenate([y[0], jnp.ones((1,), dtype=dtype)])


def _make_sc_kernel(n, d, bpt, n_words, acc_rows, rows_per_tile):
    mesh = plsc.VectorSubcoreMesh(core_axis_name="c", subcore_axis_name="s")
    epw = bpt * BLK

    @functools.partial(
        pl.kernel,
        mesh=mesh,
        compiler_params=pltpu.CompilerParams(needs_layout_passes=False),
        out_type=jax.ShapeDtypeStruct((NC, acc_rows, d), jnp.float32),
        scratch_types=[
            pltpu.VMEM((n_words,), jnp.int32),
            pltpu.VMEM((epw,), jnp.int32),
            pltpu.VMEM((4, GBLK), jnp.int32),
            pltpu.VMEM((4, GBLK), jnp.int32),
            pltpu.VMEM((4, GBLK, d), jnp.float32),
            pltpu.VMEM_SHARED((acc_rows, d), jnp.float32),
            pltpu.SemaphoreType.DMA((4,)),
            pltpu.SemaphoreType.DMA((4,)),
        ],
    )
    def sc_kernel(mask_hbm, pck_hbm, x_hbm, outp_hbm,
                  mask_v, pck_v, sstg_v, dstg_v, rows_v, acc_sh,
                  sem_g, sem_s):
        cid = lax.axis_index("c")
        sid = lax.axis_index("s")
        wid = sid * NC + cid
        trash = n + sid

        pltpu.sync_copy(mask_hbm, mask_v)
        pltpu.sync_copy(pck_hbm.at[wid], pck_v)

        zero16 = jnp.zeros((16,), jnp.float32)
        for i in range(8):
            for j in range(d // 16):
                rows_v[0, i, pl.ds(j * 16, 16)] = zero16
        base = sid * rows_per_tile

        def zero_body(k, _):
            pltpu.async_copy(rows_v.at[0, pl.ds(0, 8)],
                             acc_sh.at[pl.ds(base + k * 8, 8)], sem_g.at[0])
            return 0
        lax.fori_loop(0, rows_per_tile // 8, zero_body, 0)

        def compact_group(i, cnt):
            p16 = pck_v[pl.ds(i * 16, 16)]
            s16 = p16 & ((1 << PACK_SHIFT) - 1)
            d16 = lax.shift_right_logical(p16, PACK_SHIFT)
            ws = plsc.load_gather(mask_v, [lax.shift_right_logical(s16, 5)])
            wd = plsc.load_gather(mask_v, [lax.shift_right_logical(d16, 5)])
            bs = lax.shift_right_logical(ws, s16 & 31)
            bd = lax.shift_right_logical(wd, d16 & 31)
            keep = (bs & bd & 1) == 1
            plsc.store_compressed(pck_v.at[pl.ds(cnt, 16)], p16, mask=keep)
            return cnt + jnp.max(plsc.all_reduce_population_count(keep))

        cnt = lax.fori_loop(0, epw // 16, compact_group, jnp.int32(0), unroll=8)

        padval = jnp.full((16,), trash << PACK_SHIFT, jnp.int32)
        for g in range(BLK // 16):
            pck_v[pl.ds(cnt + g * 16, 16)] = padval
        nblk = jnp.maximum((cnt + GBLK - 1) // GBLK, 1)

        def zero_drain(k, _):
            pltpu.make_async_copy(rows_v.at[0, pl.ds(0, 8)],
                                  acc_sh.at[pl.ds(base + k * 8, 8)],
                                  sem_g.at[0]).wait()
            return 0
        lax.fori_loop(0, rows_per_tile // 8, zero_drain, 0)
        plsc.subcore_barrier()

        def stage_block(b, slot):
            for g in range(GBLK // 16):
                p16 = pck_v[pl.ds(b * GBLK + g * 16, 16)]
                sstg_v[slot, pl.ds(g * 16, 16)] = p16 & ((1 << PACK_SHIFT) - 1)
                dstg_v[slot, pl.ds(g * 16, 16)] = lax.shift_right_logical(
                    p16, PACK_SHIFT)

        for j in range(3):
            @pl.when(j < nblk)
            def _(j=j):
                stage_block(j, j)
                pltpu.make_async_copy(
                    x_hbm.at[sstg_v.at[j]], rows_v.at[j], sem_g.at[j]).start()

        def drain_scatter(k):
            s = lax.rem(k, 4)
            pltpu.make_async_copy(
                rows_v.at[s], acc_sh.at[dstg_v.at[s]], sem_s.at[s]).wait()

        def block_body(b, _):
            s = lax.rem(b, 4)
            pltpu.make_async_copy(
                x_hbm.at[sstg_v.at[s]], rows_v.at[s], sem_g.at[s]).wait()
            pltpu.async_copy(rows_v.at[s], acc_sh.at[dstg_v.at[s]],
                             sem_s.at[s], add=True)

            @pl.when(b + 3 < nblk)
            def _():
                @pl.when(b >= 1)
                def _():
                    drain_scatter(b - 1)
                sn = lax.rem(b + 3, 4)
                stage_block(b + 3, sn)
                pltpu.make_async_copy(
                    x_hbm.at[sstg_v.at[sn]], rows_v.at[sn],
                    sem_g.at[sn]).start()
            return 0

        lax.fori_loop(0, nblk, block_body, 0)

        def drain_tail(k, _):
            drain_scatter(k)
            return 0
        lax.fori_loop(jnp.maximum(nblk - 4, 0), nblk, drain_tail, 0)

        plsc.subcore_barrier()

        pltpu.sync_copy(acc_sh.at[pl.ds(base, rows_per_tile)],
                        outp_hbm.at[cid, pl.ds(base, rows_per_tile)])

    return sc_kernel


def _tc_combine(partials, n, d, blk_rows):
    def body(p_ref, o_ref):
        o_ref[...] = p_ref[0] + p_ref[1]

    return pl.pallas_call(
        body,
        grid=(n // blk_rows,),
        in_specs=[pl.BlockSpec((NC, blk_rows, d), lambda i: (0, i, 0))],
        out_specs=pl.BlockSpec((blk_rows, d), lambda i: (i, 0)),
        out_shape=jax.ShapeDtypeStruct((n, d), jnp.float32),
    )(partials)


def kernel(x, edge_index, node_mask_logit):
    n, d = x.shape
    e = edge_index.shape[1]

    hard_mask = _hard_mask(node_mask_logit, n, x.dtype)

    bpt = -(-e // (NW * BLK)) + 1
    e_pad = NW * bpt * BLK
    n_bits = -(-(n + 1) // 32) * 32
    n_words = -(-(n_bits // 32) // 16) * 16
    rows_per_tile = -(-(n + NS) // NS // 8) * 8
    acc_rows = rows_per_tile * NS

    src = edge_index[0]
    dst = edge_index[1]
    pad = e_pad - e
    src_p = jnp.concatenate([src, jnp.zeros((pad,), jnp.int32)])
    dst_p = jnp.concatenate([dst, jnp.full((pad,), n, jnp.int32)])
    packed = (src_p | (dst_p << PACK_SHIFT)).reshape(NW, bpt * BLK)

    bits = jnp.concatenate(
        [(hard_mask > 0.5).astype(jnp.uint32),
         jnp.zeros((n_words * 32 - n,), jnp.uint32)])
    words = (bits.reshape(n_words, 32) << jnp.arange(32, dtype=jnp.uint32)
             ).sum(axis=1, dtype=jnp.uint32).view(jnp.int32)

    sc = _make_sc_kernel(n, d, bpt, n_words, acc_rows, rows_per_tile)
    partials = sc(words, packed, x)

    blk_rows = 2000 if n % 2000 == 0 else 8
    return _tc_combine(partials, n, d, blk_rows)


# speedup vs baseline: 60.3417x; 1.2075x over previous
import functools

import jax
import jax.numpy as jnp
from jax import lax
from jax.experimental import pallas as pl
from jax.experimental.pallas import tpu as pltpu
from jax.experimental.pallas import tpu_sc as plsc

NC = 2
NS = 16
NW = NC * NS
BLK = 128
PACK_SHIFT = 14
GBLK = 32
SLOTS = 8
PREF = 4


def _hard_mask(node_mask_logit, n, dtype):
    node_mask_p = jax.nn.sigmoid(node_mask_logit)
    logits = jnp.log(jnp.concatenate([node_mask_p, 1.0 - node_mask_p], axis=0))
    u = jax.random.uniform(jax.random.key(42), logits.shape,
                           minval=1e-10, maxval=1.0)
    g = -jnp.log(-jnp.log(u))
    y_soft = jax.nn.softmax((logits + g) / 1.0, axis=0)
    idx = jnp.argmax(y_soft, axis=0)
    y_hard = jax.nn.one_hot(idx, logits.shape[0], axis=0, dtype=y_soft.dtype)
    y = y_hard + y_soft - jax.lax.stop_gradient(y_soft)
    return jnp.concatenate([y[0], jnp.ones((1,), dtype=dtype)])


def _make_sc_kernel(n, d, bpt, n_words, acc_rows, rows_per_tile):
    mesh = plsc.VectorSubcoreMesh(core_axis_name="c", subcore_axis_name="s")
    epw = bpt * BLK

    @functools.partial(
        pl.kernel,
        mesh=mesh,
        compiler_params=pltpu.CompilerParams(needs_layout_passes=False),
        out_type=jax.ShapeDtypeStruct((NC, acc_rows, d), jnp.float32),
        scratch_types=[
            pltpu.VMEM((n_words,), jnp.int32),
            pltpu.VMEM((epw,), jnp.int32),
            pltpu.VMEM((SLOTS, GBLK), jnp.int32),
            pltpu.VMEM((SLOTS, GBLK), jnp.int32),
            pltpu.VMEM((SLOTS, GBLK, d), jnp.float32),
            pltpu.VMEM_SHARED((acc_rows, d), jnp.float32),
            pltpu.SemaphoreType.DMA((SLOTS,)),
            pltpu.SemaphoreType.DMA((SLOTS,)),
        ],
    )
    def sc_kernel(mask_hbm, pck_hbm, x_hbm, outp_hbm,
                  mask_v, pck_v, sstg_v, dstg_v, rows_v, acc_sh,
                  sem_g, sem_s):
        cid = lax.axis_index("c")
        sid = lax.axis_index("s")
        wid = sid * NC + cid
        trash = n + sid

        pltpu.sync_copy(mask_hbm, mask_v)
        pltpu.sync_copy(pck_hbm.at[wid], pck_v)

        zero16 = jnp.zeros((16,), jnp.float32)
        for i in range(8):
            for j in range(d // 16):
                rows_v[0, i, pl.ds(j * 16, 16)] = zero16
        base = sid * rows_per_tile

        def zero_body(k, _):
            pltpu.async_copy(rows_v.at[0, pl.ds(0, 8)],
                             acc_sh.at[pl.ds(base + k * 8, 8)], sem_g.at[0])
            return 0
        lax.fori_loop(0, rows_per_tile // 8, zero_body, 0)

        def compact_group(i, cnt):
            p16 = pck_v[pl.ds(i * 16, 16)]
            s16 = p16 & ((1 << PACK_SHIFT) - 1)
            d16 = lax.shift_right_logical(p16, PACK_SHIFT)
            ws = plsc.load_gather(mask_v, [lax.shift_right_logical(s16, 5)])
            wd = plsc.load_gather(mask_v, [lax.shift_right_logical(d16, 5)])
            bs = lax.shift_right_logical(ws, s16 & 31)
            bd = lax.shift_right_logical(wd, d16 & 31)
            keep = (bs & bd & 1) == 1
            plsc.store_compressed(pck_v.at[pl.ds(cnt, 16)], p16, mask=keep)
            return cnt + jnp.max(plsc.all_reduce_population_count(keep))

        cnt = lax.fori_loop(0, epw // 16, compact_group, jnp.int32(0), unroll=8)

        padval = jnp.full((16,), trash << PACK_SHIFT, jnp.int32)
        for g in range(BLK // 16):
            pck_v[pl.ds(cnt + g * 16, 16)] = padval
        nblk = jnp.maximum((cnt + GBLK - 1) // GBLK, 1)

        def zero_drain(k, _):
            pltpu.make_async_copy(rows_v.at[0, pl.ds(0, 8)],
                                  acc_sh.at[pl.ds(base + k * 8, 8)],
                                  sem_g.at[0]).wait()
            return 0
        lax.fori_loop(0, rows_per_tile // 8, zero_drain, 0)
        plsc.subcore_barrier()

        def stage_block(b, slot):
            for g in range(GBLK // 16):
                p16 = pck_v[pl.ds(b * GBLK + g * 16, 16)]
                sstg_v[slot, pl.ds(g * 16, 16)] = p16 & ((1 << PACK_SHIFT) - 1)
                dstg_v[slot, pl.ds(g * 16, 16)] = lax.shift_right_logical(
                    p16, PACK_SHIFT)

        for j in range(PREF):
            @pl.when(j < nblk)
            def _(j=j):
                stage_block(j, j)
                pltpu.make_async_copy(
                    x_hbm.at[sstg_v.at[j]], rows_v.at[j], sem_g.at[j]).start()

        def drain_scatter(k):
            s = lax.rem(k, SLOTS)
            pltpu.make_async_copy(
                rows_v.at[s], acc_sh.at[dstg_v.at[s]], sem_s.at[s]).wait()

        def block_body(b, _):
            s = lax.rem(b, SLOTS)
            pltpu.make_async_copy(
                x_hbm.at[sstg_v.at[s]], rows_v.at[s], sem_g.at[s]).wait()
            pltpu.async_copy(rows_v.at[s], acc_sh.at[dstg_v.at[s]],
                             sem_s.at[s], add=True)

            @pl.when(b + PREF < nblk)
            def _():
                @pl.when(b >= SLOTS - PREF)
                def _():
                    drain_scatter(b - (SLOTS - PREF))
                sn = lax.rem(b + PREF, SLOTS)
                stage_block(b + PREF, sn)
                pltpu.make_async_copy(
                    x_hbm.at[sstg_v.at[sn]], rows_v.at[sn],
                    sem_g.at[sn]).start()
            return 0

        lax.fori_loop(0, nblk, block_body, 0)

        def drain_tail(k, _):
            drain_scatter(k)
            return 0
        lax.fori_loop(jnp.maximum(nblk - SLOTS, 0), nblk, drain_tail, 0)

        plsc.subcore_barrier()

        pltpu.sync_copy(acc_sh.at[pl.ds(base, rows_per_tile)],
                        outp_hbm.at[cid, pl.ds(base, rows_per_tile)])

    return sc_kernel


def _tc_combine(partials, n, d, blk_rows):
    def body(p_ref, o_ref):
        o_ref[...] = p_ref[0] + p_ref[1]

    return pl.pallas_call(
        body,
        grid=(n // blk_rows,),
        in_specs=[pl.BlockSpec((NC, blk_rows, d), lambda i: (0, i, 0))],
        out_specs=pl.BlockSpec((blk_rows, d), lambda i: (i, 0)),
        out_shape=jax.ShapeDtypeStruct((n, d), jnp.float32),
    )(partials)


def kernel(x, edge_index, node_mask_logit):
    n, d = x.shape
    e = edge_index.shape[1]

    hard_mask = _hard_mask(node_mask_logit, n, x.dtype)

    bpt = -(-e // (NW * BLK)) + 1
    e_pad = NW * bpt * BLK
    n_bits = -(-(n + 1) // 32) * 32
    n_words = -(-(n_bits // 32) // 16) * 16
    rows_per_tile = -(-(n + NS) // NS // 8) * 8
    acc_rows = rows_per_tile * NS

    src = edge_index[0]
    dst = edge_index[1]
    pad = e_pad - e
    src_p = jnp.concatenate([src, jnp.zeros((pad,), jnp.int32)])
    dst_p = jnp.concatenate([dst, jnp.full((pad,), n, jnp.int32)])
    packed = (src_p | (dst_p << PACK_SHIFT)).reshape(NW, bpt * BLK)

    bits = jnp.concatenate(
        [(hard_mask > 0.5).astype(jnp.uint32),
         jnp.zeros((n_words * 32 - n,), jnp.uint32)])
    words = (bits.reshape(n_words, 32) << jnp.arange(32, dtype=jnp.uint32)
             ).sum(axis=1, dtype=jnp.uint32).view(jnp.int32)

    sc = _make_sc_kernel(n, d, bpt, n_words, acc_rows, rows_per_tile)
    partials = sc(words, packed, x)

    blk_rows = 2000 if n % 2000 == 0 else 8
    return _tc_combine(partials, n, d, blk_rows)


# speedup vs baseline: 64.9682x; 1.0767x over previous
import functools

import jax
import jax.numpy as jnp
from jax import lax
from jax.experimental import pallas as pl
from jax.experimental.pallas import tpu as pltpu
from jax.experimental.pallas import tpu_sc as plsc

NC = 2
NS = 16
NW = NC * NS
PACK_SHIFT = 14
GBLK = 32
SLOTS = 8
PREF = 4
SBLK = 128


def _hard_mask(node_mask_logit, n, dtype):
    node_mask_p = jax.nn.sigmoid(node_mask_logit)
    logits = jnp.log(jnp.concatenate([node_mask_p, 1.0 - node_mask_p], axis=0))
    u = jax.random.uniform(jax.random.key(42), logits.shape,
                           minval=1e-10, maxval=1.0)
    g = -jnp.log(-jnp.log(u))
    y_soft = jax.nn.softmax((logits + g) / 1.0, axis=0)
    idx = jnp.argmax(y_soft, axis=0)
    y_hard = jax.nn.one_hot(idx, logits.shape[0], axis=0, dtype=y_soft.dtype)
    y = y_hard + y_soft - jax.lax.stop_gradient(y_soft)
    return jnp.concatenate([y[0], jnp.ones((1,), dtype=dtype)])


def _make_sc_kernel(n, d, ept, n_words, acc_rows, rows_per_tile):
    mesh = plsc.VectorSubcoreMesh(core_axis_name="c", subcore_axis_name="s")
    epw = ept + SBLK

    @functools.partial(
        pl.kernel,
        mesh=mesh,
        compiler_params=pltpu.CompilerParams(needs_layout_passes=False),
        out_type=jax.ShapeDtypeStruct((NC, acc_rows, d), jnp.float32),
        scratch_types=[
            pltpu.VMEM((n_words,), jnp.int32),
            pltpu.VMEM((epw,), jnp.int32),
            pltpu.VMEM((SLOTS, GBLK), jnp.int32),
            pltpu.VMEM((SLOTS, GBLK), jnp.int32),
            pltpu.VMEM((8, d), jnp.float32),
            pltpu.VMEM_SHARED((acc_rows, d), jnp.float32),
            pltpu.SemaphoreType.DMA((SLOTS,)),
            pltpu.SemaphoreType.DMA((SLOTS,)),
        ],
    )
    def sc_kernel(mask_hbm, ei_hbm, x_hbm, outp_hbm,
                  mask_v, pck_v, sstg_v, dstg_v, zero_v, acc_sh,
                  sem_g, sem_s):
        cid = lax.axis_index("c")
        sid = lax.axis_index("s")
        wid = sid * NC + cid
        trash = n + sid

        pltpu.sync_copy(mask_hbm, mask_v)

        zero16 = jnp.zeros((16,), jnp.float32)
        for i in range(8):
            for j in range(d // 16):
                zero_v[i, pl.ds(j * 16, 16)] = zero16
        base = sid * rows_per_tile

        def zero_body(k, _):
            pltpu.async_copy(zero_v, acc_sh.at[pl.ds(base + k * 8, 8)],
                             sem_g.at[0])
            return 0
        lax.fori_loop(0, rows_per_tile // 8, zero_body, 0)

        def phase_a(dst_v):
            pltpu.sync_copy(ei_hbm.at[0, wid], pck_v.at[pl.ds(0, ept)])
            pltpu.sync_copy(ei_hbm.at[1, wid], dst_v)

            def compact_group(i, cnt):
                s16 = pck_v[pl.ds(i * 16, 16)]
                d16 = dst_v[pl.ds(i * 16, 16)]
                ws = plsc.load_gather(
                    mask_v, [lax.shift_right_logical(s16, 5)])
                wd = plsc.load_gather(
                    mask_v, [lax.shift_right_logical(d16, 5)])
                bs = lax.shift_right_logical(ws, s16 & 31)
                bd = lax.shift_right_logical(wd, d16 & 31)
                keep = (bs & bd & 1) == 1
                p16 = s16 | (d16 << PACK_SHIFT)
                plsc.store_compressed(pck_v.at[pl.ds(cnt, 16)], p16,
                                      mask=keep)
                return cnt + jnp.max(plsc.all_reduce_population_count(keep))

            return lax.fori_loop(0, ept // 16, compact_group, jnp.int32(0),
                                 unroll=8)

        cnt = pl.run_scoped(phase_a, pltpu.VMEM((ept,), jnp.int32))

        padval = jnp.full((16,), trash << PACK_SHIFT, jnp.int32)
        for g in range(SBLK // 16):
            pck_v[pl.ds(cnt + g * 16, 16)] = padval
        nblk = jnp.maximum((cnt + GBLK - 1) // GBLK, 1)

        def zero_drain(k, _):
            pltpu.make_async_copy(zero_v, acc_sh.at[pl.ds(base + k * 8, 8)],
                                  sem_g.at[0]).wait()
            return 0
        lax.fori_loop(0, rows_per_tile // 8, zero_drain, 0)
        plsc.subcore_barrier()

        def phase_b(rows_v):
            def stage_block(b, slot):
                for g in range(GBLK // 16):
                    p16 = pck_v[pl.ds(b * GBLK + g * 16, 16)]
                    sstg_v[slot, pl.ds(g * 16, 16)] = (
                        p16 & ((1 << PACK_SHIFT) - 1))
                    dstg_v[slot, pl.ds(g * 16, 16)] = lax.shift_right_logical(
                        p16, PACK_SHIFT)

            for j in range(PREF):
                @pl.when(j < nblk)
                def _(j=j):
                    stage_block(j, j)
                    pltpu.make_async_copy(
                        x_hbm.at[sstg_v.at[j]], rows_v.at[j],
                        sem_g.at[j]).start()

            def drain_scatter(k):
                s = lax.rem(k, SLOTS)
                pltpu.make_async_copy(
                    rows_v.at[s], acc_sh.at[dstg_v.at[s]], sem_s.at[s]).wait()

            def block_body(b, _):
                s = lax.rem(b, SLOTS)
                pltpu.make_async_copy(
                    x_hbm.at[sstg_v.at[s]], rows_v.at[s], sem_g.at[s]).wait()
                pltpu.async_copy(rows_v.at[s], acc_sh.at[dstg_v.at[s]],
                                 sem_s.at[s], add=True)

                @pl.when(b + PREF < nblk)
                def _():
                    @pl.when(b >= SLOTS - PREF)
                    def _():
                        drain_scatter(b - (SLOTS - PREF))
                    sn = lax.rem(b + PREF, SLOTS)
                    stage_block(b + PREF, sn)
                    pltpu.make_async_copy(
                        x_hbm.at[sstg_v.at[sn]], rows_v.at[sn],
                        sem_g.at[sn]).start()
                return 0

            lax.fori_loop(0, nblk, block_body, 0)

            def drain_tail(k, _):
                drain_scatter(k)
                return 0
            lax.fori_loop(jnp.maximum(nblk - SLOTS, 0), nblk, drain_tail, 0)

        pl.run_scoped(phase_b, pltpu.VMEM((SLOTS, GBLK, d), jnp.float32))
        plsc.subcore_barrier()

        pltpu.sync_copy(acc_sh.at[pl.ds(base, rows_per_tile)],
                        outp_hbm.at[cid, pl.ds(base, rows_per_tile)])

    return sc_kernel


def _tc_combine(partials, n, d, blk_rows):
    def body(p_ref, o_ref):
        o_ref[...] = p_ref[0] + p_ref[1]

    return pl.pallas_call(
        body,
        grid=(n // blk_rows,),
        in_specs=[pl.BlockSpec((NC, blk_rows, d), lambda i: (0, i, 0))],
        out_specs=pl.BlockSpec((blk_rows, d), lambda i: (i, 0)),
        out_shape=jax.ShapeDtypeStruct((n, d), jnp.float32),
    )(partials)


def kernel(x, edge_index, node_mask_logit):
    n, d = x.shape
    e = edge_index.shape[1]

    hard_mask = _hard_mask(node_mask_logit, n, x.dtype)

    n_bits = -(-(n + 1) // 32) * 32
    n_words = -(-(n_bits // 32) // 16) * 16
    rows_per_tile = -(-(n + NS) // NS // 8) * 8
    acc_rows = rows_per_tile * NS

    ei = edge_index
    if e % (NW * 128) != 0:
        pad = NW * 128 - e % (NW * 128)
        ei = jnp.concatenate(
            [ei, jnp.broadcast_to(jnp.array([[0], [n]], jnp.int32),
                                  (2, pad))], axis=1)
        e += pad
    ept = e // NW
    ei = ei.reshape(2, NW, ept)

    bits = jnp.concatenate(
        [(hard_mask > 0.5).astype(jnp.uint32),
         jnp.zeros((n_words * 32 - n,), jnp.uint32)])
    words = (bits.reshape(n_words, 32) << jnp.arange(32, dtype=jnp.uint32)
             ).sum(axis=1, dtype=jnp.uint32).view(jnp.int32)

    sc = _make_sc_kernel(n, d, ept, n_words, acc_rows, rows_per_tile)
    partials = sc(words, ei, x)

    blk_rows = 2000 if n % 2000 == 0 else 8
    return _tc_combine(partials, n, d, blk_rows)


# speedup vs baseline: 65.0437x; 1.0012x over previous
import functools

import jax
import jax.numpy as jnp
from jax import lax
from jax.experimental import pallas as pl
from jax.experimental.pallas import tpu as pltpu
from jax.experimental.pallas import tpu_sc as plsc

NC = 2
NS = 16
NW = NC * NS
PACK_SHIFT = 14
GBLK = 32
SLOTS = 8
PREF = 4
SBLK = 128


def _hard_mask(node_mask_logit, n, dtype):
    node_mask_p = jax.nn.sigmoid(node_mask_logit)
    logits = jnp.log(jnp.concatenate([node_mask_p, 1.0 - node_mask_p], axis=0))
    u = jax.random.uniform(jax.random.key(42), logits.shape,
                           minval=1e-10, maxval=1.0)
    g = -jnp.log(-jnp.log(u))
    y_soft = jax.nn.softmax((logits + g) / 1.0, axis=0)
    idx = jnp.argmax(y_soft, axis=0)
    y_hard = jax.nn.one_hot(idx, logits.shape[0], axis=0, dtype=y_soft.dtype)
    y = y_hard + y_soft - jax.lax.stop_gradient(y_soft)
    return jnp.concatenate([y[0], jnp.ones((1,), dtype=dtype)])


def _make_sc_kernel(n, d, ept, n_words, acc_rows, rows_per_tile):
    mesh = plsc.VectorSubcoreMesh(core_axis_name="c", subcore_axis_name="s")
    epw = ept + SBLK

    @functools.partial(
        pl.kernel,
        mesh=mesh,
        compiler_params=pltpu.CompilerParams(needs_layout_passes=False),
        out_type=jax.ShapeDtypeStruct((NC, acc_rows, d), jnp.float32),
        scratch_types=[
            pltpu.VMEM((n_words,), jnp.int32),
            pltpu.VMEM((epw,), jnp.int32),
            pltpu.VMEM((SLOTS, GBLK), jnp.int32),
            pltpu.VMEM((SLOTS, GBLK), jnp.int32),
            pltpu.VMEM((8, d), jnp.float32),
            pltpu.VMEM_SHARED((acc_rows, d), jnp.float32),
            pltpu.SemaphoreType.DMA((SLOTS,)),
            pltpu.SemaphoreType.DMA((SLOTS,)),
        ],
    )
    def sc_kernel(mask_hbm, ei_hbm, x_hbm, outp_hbm,
                  mask_v, pck_v, sstg_v, dstg_v, zero_v, acc_sh,
                  sem_g, sem_s):
        cid = lax.axis_index("c")
        sid = lax.axis_index("s")
        wid = sid * NC + cid
        trash = n + sid

        pltpu.sync_copy(mask_hbm, mask_v)

        zero16 = jnp.zeros((16,), jnp.float32)
        for i in range(8):
            for j in range(d // 16):
                zero_v[i, pl.ds(j * 16, 16)] = zero16
        base = sid * rows_per_tile

        def zero_body(k, _):
            pltpu.async_copy(zero_v, acc_sh.at[pl.ds(base + k * 8, 8)],
                             sem_g.at[0])
            return 0
        lax.fori_loop(0, rows_per_tile // 8, zero_body, 0)

        def phase_a(dst_v):
            pltpu.sync_copy(ei_hbm.at[0, wid], pck_v.at[pl.ds(0, ept)])
            pltpu.sync_copy(ei_hbm.at[1, wid], dst_v)

            def compact_group(i, cnt):
                s16 = pck_v[pl.ds(i * 16, 16)]
                d16 = dst_v[pl.ds(i * 16, 16)]
                ws = plsc.load_gather(
                    mask_v, [lax.shift_right_logical(s16, 5)])
                wd = plsc.load_gather(
                    mask_v, [lax.shift_right_logical(d16, 5)])
                bs = lax.shift_right_logical(ws, s16 & 31)
                bd = lax.shift_right_logical(wd, d16 & 31)
                keep = (bs & bd & 1) == 1
                p16 = s16 | (d16 << PACK_SHIFT)
                plsc.store_compressed(pck_v.at[pl.ds(cnt, 16)], p16,
                                      mask=keep)
                return cnt + plsc.all_reduce_population_count(keep)[0]

            return lax.fori_loop(0, ept // 16, compact_group, jnp.int32(0),
                                 unroll=8)

        cnt = pl.run_scoped(phase_a, pltpu.VMEM((ept,), jnp.int32))

        padval = jnp.full((16,), trash << PACK_SHIFT, jnp.int32)
        for g in range(SBLK // 16):
            pck_v[pl.ds(cnt + g * 16, 16)] = padval
        nblk = jnp.maximum((cnt + GBLK - 1) // GBLK, 1)

        def zero_drain(k, _):
            pltpu.make_async_copy(zero_v, acc_sh.at[pl.ds(base + k * 8, 8)],
                                  sem_g.at[0]).wait()
            return 0
        lax.fori_loop(0, rows_per_tile // 8, zero_drain, 0)
        plsc.subcore_barrier()

        def phase_b(rows_v):
            def stage_block(b, slot):
                for g in range(GBLK // 16):
                    p16 = pck_v[pl.ds(b * GBLK + g * 16, 16)]
                    sstg_v[slot, pl.ds(g * 16, 16)] = (
                        p16 & ((1 << PACK_SHIFT) - 1))
                    dstg_v[slot, pl.ds(g * 16, 16)] = lax.shift_right_logical(
                        p16, PACK_SHIFT)

            for j in range(PREF):
                @pl.when(j < nblk)
                def _(j=j):
                    stage_block(j, j)
                    pltpu.make_async_copy(
                        x_hbm.at[sstg_v.at[j]], rows_v.at[j],
                        sem_g.at[j]).start()

            def drain_scatter(k):
                s = lax.rem(k, SLOTS)
                pltpu.make_async_copy(
                    rows_v.at[s], acc_sh.at[dstg_v.at[s]], sem_s.at[s]).wait()

            def block_body(b, _):
                s = lax.rem(b, SLOTS)
                pltpu.make_async_copy(
                    x_hbm.at[sstg_v.at[s]], rows_v.at[s], sem_g.at[s]).wait()
                pltpu.async_copy(rows_v.at[s], acc_sh.at[dstg_v.at[s]],
                                 sem_s.at[s], add=True)

                @pl.when(b + PREF < nblk)
                def _():
                    @pl.when(b >= SLOTS - PREF)
                    def _():
                        drain_scatter(b - (SLOTS - PREF))
                    sn = lax.rem(b + PREF, SLOTS)
                    stage_block(b + PREF, sn)
                    pltpu.make_async_copy(
                        x_hbm.at[sstg_v.at[sn]], rows_v.at[sn],
                        sem_g.at[sn]).start()
                return 0

            lax.fori_loop(0, nblk, block_body, 0)

            def drain_tail(k, _):
                drain_scatter(k)
                return 0
            lax.fori_loop(jnp.maximum(nblk - SLOTS, 0), nblk, drain_tail, 0)

        pl.run_scoped(phase_b, pltpu.VMEM((SLOTS, GBLK, d), jnp.float32))
        plsc.subcore_barrier()

        pltpu.sync_copy(acc_sh.at[pl.ds(base, rows_per_tile)],
                        outp_hbm.at[cid, pl.ds(base, rows_per_tile)])

    return sc_kernel


def _tc_combine(partials, n, d, blk_rows):
    def body(p_ref, o_ref):
        o_ref[...] = p_ref[0] + p_ref[1]

    return pl.pallas_call(
        body,
        grid=(n // blk_rows,),
        in_specs=[pl.BlockSpec((NC, blk_rows, d), lambda i: (0, i, 0))],
        out_specs=pl.BlockSpec((blk_rows, d), lambda i: (i, 0)),
        out_shape=jax.ShapeDtypeStruct((n, d), jnp.float32),
    )(partials)


def kernel(x, edge_index, node_mask_logit):
    n, d = x.shape
    e = edge_index.shape[1]

    hard_mask = _hard_mask(node_mask_logit, n, x.dtype)

    n_bits = -(-(n + 1) // 32) * 32
    n_words = -(-(n_bits // 32) // 16) * 16
    rows_per_tile = -(-(n + NS) // NS // 8) * 8
    acc_rows = rows_per_tile * NS

    ei = edge_index
    if e % (NW * 128) != 0:
        pad = NW * 128 - e % (NW * 128)
        ei = jnp.concatenate(
            [ei, jnp.broadcast_to(jnp.array([[0], [n]], jnp.int32),
                                  (2, pad))], axis=1)
        e += pad
    ept = e // NW
    ei = ei.reshape(2, NW, ept)

    bits = jnp.concatenate(
        [(hard_mask > 0.5).astype(jnp.uint32),
         jnp.zeros((n_words * 32 - n,), jnp.uint32)])
    words = (bits.reshape(n_words, 32) << jnp.arange(32, dtype=jnp.uint32)
             ).sum(axis=1, dtype=jnp.uint32).view(jnp.int32)

    sc = _make_sc_kernel(n, d, ept, n_words, acc_rows, rows_per_tile)
    partials = sc(words, ei, x)

    blk_rows = 2000 if n % 2000 == 0 else 8
    return _tc_combine(partials, n, d, blk_rows)
